# baseline (speedup 1.0000x reference)
import functools

import jax
import jax.numpy as jnp
from jax import lax
from jax.experimental import pallas as pl
from jax.experimental.pallas import tpu as pltpu
from jax.experimental.pallas import tpu_sc as plsc

N = 10000
E = 320000
D = 128
H = 128
G = 64

CH = 128
NC = 2
NS = 16
NW = NC * NS
CPT = 80
EC = NW * CPT
EP = EC * CH
NBUF = 4
NP = 10240
RPT = NP // NS
BM = 2000
NB = N // BM

_f32 = jnp.float32


def _sc_mesh():
    return plsc.VectorSubcoreMesh(core_axis_name="c", subcore_axis_name="s")


def _sc_agg_body(src_hbm, dst_hbm, z_hbm, zz_hbm, out_hbm,
                 s0, s1, d0, d1, rows_v, acc_sh, gsem, ia, ib):
    c = lax.axis_index("c")
    s = lax.axis_index("s")
    w = s * NC + c
    r0 = s * RPT
    sbuf = [s0, s1]
    dbuf = [d0, d1]
    isem = [ia, ib]

    pltpu.sync_copy(zz_hbm.at[pl.ds(r0, RPT)], acc_sh.at[pl.ds(r0, RPT)])

    def idx_start(j, p):
        cid = w + j * NW
        pltpu.async_copy(src_hbm.at[cid], sbuf[p], isem[p])
        pltpu.async_copy(dst_hbm.at[cid], dbuf[p], isem[p])

    def idx_wait(p):
        pltpu.make_async_copy(src_hbm.at[w], sbuf[p], isem[p]).wait()
        pltpu.make_async_copy(dst_hbm.at[w], dbuf[p], isem[p]).wait()

    idx_start(0, 0)
    plsc.subcore_barrier()

    def outer(jo, carry):
        for p in (0, 1):
            j = jo * 2 + p
            idx_wait(p)
            jn = jnp.where(j + 1 < CPT, j + 1, 0)
            idx_start(jn, 1 - p)
            pltpu.async_copy(z_hbm.at[sbuf[p]], rows_v, gsem).wait()
            pltpu.sync_copy(rows_v, acc_sh.at[dbuf[p]], add=True)

        return carry

    lax.fori_loop(0, CPT // 2, outer, 0)
    idx_wait(0)
    plsc.subcore_barrier()

    pltpu.sync_copy(acc_sh.at[pl.ds(r0, RPT)],
                    out_hbm.at[c, pl.ds(r0, RPT)])


def _make_sc_agg():
    return functools.partial(
        pl.kernel,
        mesh=_sc_mesh(),
        out_type=jax.ShapeDtypeStruct((NC, NP, H), _f32),
        scratch_types=[
            pltpu.VMEM((CH,), jnp.int32),
            pltpu.VMEM((CH,), jnp.int32),
            pltpu.VMEM((CH,), jnp.int32),
            pltpu.VMEM((CH,), jnp.int32),
            pltpu.VMEM((CH, H), _f32),
            pltpu.VMEM_SHARED((NP, H), _f32),
            pltpu.SemaphoreType.DMA,
            pltpu.SemaphoreType.DMA,
            pltpu.SemaphoreType.DMA,
        ],
    )(_sc_agg_body)


def _sc_cnt_body(dst_hbm, ones_hbm, zz_hbm, out_hbm,
                 d0, d1, ones_v, acc_sh, ia, ib):
    c = lax.axis_index("c")
    s = lax.axis_index("s")
    w = s * NC + c
    r0 = s * RPT
    dbuf = [d0, d1]
    isem = [ia, ib]

    pltpu.sync_copy(zz_hbm.at[pl.ds(r0, RPT)], acc_sh.at[pl.ds(r0, RPT)])
    pltpu.sync_copy(ones_hbm, ones_v)

    def idx_start(j, p):
        pltpu.async_copy(dst_hbm.at[w + j * NW], dbuf[p], isem[p])

    def idx_wait(j, p):
        pltpu.make_async_copy(dst_hbm.at[w + j * NW], dbuf[p], isem[p]).wait()

    idx_start(0, 0)
    plsc.subcore_barrier()

    def outer(jo, carry):
        for p in (0, 1):
            j = jo * 2 + p
            idx_wait(j, p)

            @pl.when(j + 1 < CPT)
            def _():
                idx_start(j + 1, 1 - p)

            pltpu.sync_copy(ones_v, acc_sh.at[dbuf[p]], add=True)

        return carry

    lax.fori_loop(0, CPT // 2, outer, 0)
    plsc.subcore_barrier()

    pltpu.sync_copy(acc_sh.at[pl.ds(r0, RPT)],
                    out_hbm.at[c, pl.ds(r0, RPT)])


def _make_sc_cnt():
    return functools.partial(
        pl.kernel,
        mesh=_sc_mesh(),
        out_type=jax.ShapeDtypeStruct((NC, NP, H), _f32),
        scratch_types=[
            pltpu.VMEM((CH,), jnp.int32),
            pltpu.VMEM((CH,), jnp.int32),
            pltpu.VMEM((CH, H), _f32),
            pltpu.VMEM_SHARED((NP, H), _f32),
            pltpu.SemaphoreType.DMA,
            pltpu.SemaphoreType.DMA,
        ],
    )(_sc_cnt_body)


def _recip(c):
    r = 1.0 / c
    r = r * (2.0 - c * r)
    r = r * (2.0 - c * r)
    return r


def _layer_body(p_ref, cnt_ref, xp_ref, wl_ref, bl_ref, wr_ref, h_ref):
    inv = _recip(jnp.maximum(cnt_ref[0] + cnt_ref[1], 1.0))
    mean = (p_ref[0] + p_ref[1]) * inv
    h_ref[...] = jnp.maximum(
        jnp.dot(mean, wl_ref[...], preferred_element_type=_f32) +
        bl_ref[...] +
        jnp.dot(xp_ref[...], wr_ref[...], preferred_element_type=_f32), 0.0)


def _pool_body(h_ref, b_ref, wg_ref, bg_ref, w1_ref, b1_ref, w2_ref,
               b2_ref, o_ref):
    h = h_ref[...]
    gate = jnp.dot(h, wg_ref[...], preferred_element_type=_f32) + bg_ref[...]
    mask = b_ref[...] == lax.broadcasted_iota(jnp.int32, (N, G), 1)
    mg = jnp.max(jnp.where(mask, gate, -3e38), axis=0, keepdims=True)
    ew = jnp.where(mask, jnp.exp(gate - mg), 0.0)
    dn = jnp.sum(ew, axis=0, keepdims=True)
    alpha = ew * _recip(jnp.where(dn > 0.0, dn, 1.0))
    pooled = lax.dot_general(alpha, h, (((0,), (0,)), ((), ())),
                             preferred_element_type=_f32,
                             precision=lax.Precision.HIGHEST)
    t = jnp.maximum(
        jnp.dot(pooled, w1_ref[...], preferred_element_type=_f32) +
        b1_ref[...], 0.0)
    o_ref[...] = jnp.dot(t, w2_ref[...], preferred_element_type=_f32) + b2_ref[...]


def _row_spec(width):
    return pl.BlockSpec((BM, width), lambda i: (i, 0))


def _rep_spec(shape):
    nd = len(shape)
    return pl.BlockSpec(shape, lambda i: (0,) * nd)


def _tc_layer(p, cnt2d, xp, wl, bl, wr):
    return pl.pallas_call(
        _layer_body,
        grid=(NB,),
        in_specs=[
            pl.BlockSpec((NC, BM, H), lambda i: (0, i, 0)),
            pl.BlockSpec((NC, BM, 1), lambda i: (0, i, 0)),
            _row_spec(H), _rep_spec((H, H)), _rep_spec((1, H)),
            _rep_spec((H, H)),
        ],
        out_specs=_row_spec(H),
        out_shape=jax.ShapeDtypeStruct((N, H), _f32),
    )(p, cnt2d, xp, wl, bl, wr)


def _tc_pool(h, batch2d, wg, bg, w1, b1, w2, b2):
    return pl.pallas_call(
        _pool_body,
        out_shape=jax.ShapeDtypeStruct((G, 1), _f32),
    )(h, batch2d, wg, bg, w1, b1, w2, b2)


def kernel(x, edge_index, batch, Wl1, bl1, Wr1, Wl2, bl2, Wr2, Wl3, bl3, Wr3,
           Wg, bg, W1, b1, W2, b2):
    pad = EP - E
    src2d = jnp.concatenate(
        [edge_index[0], jnp.zeros((pad,), jnp.int32)]).reshape(EC, CH)
    dst2d = jnp.concatenate(
        [edge_index[1], jnp.full((pad,), N, jnp.int32)]).reshape(EC, CH)
    batch2d = batch.reshape(N, 1)
    zz = jnp.zeros((NP, H), _f32)

    sc_agg = _make_sc_agg()
    cnt2d = _make_sc_cnt()(dst2d, jnp.ones((CH, H), _f32), zz)[:, :, :1]

    p1 = sc_agg(src2d, dst2d, x, zz)
    h1 = _tc_layer(p1, cnt2d, x, Wl1, bl1.reshape(1, H), Wr1)
    p2 = sc_agg(src2d, dst2d, h1, zz)
    h2 = _tc_layer(p2, cnt2d, h1, Wl2, bl2.reshape(1, H), Wr2)
    p3 = sc_agg(src2d, dst2d, h2, zz)
    h3 = _tc_layer(p3, cnt2d, h2, Wl3, bl3.reshape(1, H), Wr3)
    out = _tc_pool(h3, batch2d, Wg, bg.reshape(1, 1),
                   W1, b1.reshape(1, H), W2, b2.reshape(1, 1))
    return out


# speedup vs baseline: 1.0083x; 1.0083x over previous
import functools

import jax
import jax.numpy as jnp
from jax import lax
from jax.experimental import pallas as pl
from jax.experimental.pallas import tpu as pltpu
from jax.experimental.pallas import tpu_sc as plsc

N = 10000
E = 320000
D = 128
H = 128
G = 64

CH = 128
NC = 2
NS = 16
NW = NC * NS
CPT = 80
EC = NW * CPT
EP = EC * CH
NBUF = 4
NP = 10240
RPT = NP // NS
BM = 2000
NB = N // BM

_f32 = jnp.float32


def _sc_mesh():
    return plsc.VectorSubcoreMesh(core_axis_name="c", subcore_axis_name="s")


def _sc_agg_body(src_hbm, dst_hbm, z_hbm, zz_hbm, out_hbm,
                 s0, s1, d0, d1, ra, rb, acc_sh, isem, gsem, ssem):
    c = lax.axis_index("c")
    s = lax.axis_index("s")
    w = s * NC + c
    r0 = s * RPT

    pltpu.sync_copy(zz_hbm.at[pl.ds(r0, RPT)], acc_sh.at[pl.ds(r0, RPT)])
    plsc.subcore_barrier()

    def body(jo, carry):
        ca = w + (2 * jo) * NW
        cb = w + (2 * jo + 1) * NW
        h1 = pltpu.async_copy(src_hbm.at[ca], s0, isem)
        h2 = pltpu.async_copy(dst_hbm.at[ca], d0, isem)
        h3 = pltpu.async_copy(src_hbm.at[cb], s1, isem)
        h4 = pltpu.async_copy(dst_hbm.at[cb], d1, isem)
        h1.wait(); h2.wait(); h3.wait(); h4.wait()
        g1 = pltpu.async_copy(z_hbm.at[s0], ra, gsem)
        g2 = pltpu.async_copy(z_hbm.at[s1], rb, gsem)
        g1.wait(); g2.wait()
        t1 = pltpu.async_copy(ra, acc_sh.at[d0], ssem, add=True)
        t2 = pltpu.async_copy(rb, acc_sh.at[d1], ssem, add=True)
        t1.wait(); t2.wait()
        return carry

    lax.fori_loop(0, CPT // 2, body, 0)
    plsc.subcore_barrier()

    pltpu.sync_copy(acc_sh.at[pl.ds(r0, RPT)],
                    out_hbm.at[c, pl.ds(r0, RPT)])


def _make_sc_agg():
    return functools.partial(
        pl.kernel,
        mesh=_sc_mesh(),
        out_type=jax.ShapeDtypeStruct((NC, NP, H), _f32),
        scratch_types=[
            pltpu.VMEM((CH,), jnp.int32),
            pltpu.VMEM((CH,), jnp.int32),
            pltpu.VMEM((CH,), jnp.int32),
            pltpu.VMEM((CH,), jnp.int32),
            pltpu.VMEM((CH, H), _f32),
            pltpu.VMEM((CH, H), _f32),
            pltpu.VMEM_SHARED((NP, H), _f32),
            pltpu.SemaphoreType.DMA,
            pltpu.SemaphoreType.DMA,
            pltpu.SemaphoreType.DMA,
        ],
    )(_sc_agg_body)


def _sc_cnt_body(dst_hbm, ones_hbm, zz_hbm, out_hbm,
                 d0, d1, ones_v, acc_sh, ia, ib):
    c = lax.axis_index("c")
    s = lax.axis_index("s")
    w = s * NC + c
    r0 = s * RPT
    dbuf = [d0, d1]
    isem = [ia, ib]

    pltpu.sync_copy(zz_hbm.at[pl.ds(r0, RPT)], acc_sh.at[pl.ds(r0, RPT)])
    pltpu.sync_copy(ones_hbm, ones_v)

    def idx_start(j, p):
        pltpu.async_copy(dst_hbm.at[w + j * NW], dbuf[p], isem[p])

    def idx_wait(j, p):
        pltpu.make_async_copy(dst_hbm.at[w + j * NW], dbuf[p], isem[p]).wait()

    idx_start(0, 0)
    plsc.subcore_barrier()

    def outer(jo, carry):
        for p in (0, 1):
            j = jo * 2 + p
            idx_wait(j, p)

            @pl.when(j + 1 < CPT)
            def _():
                idx_start(j + 1, 1 - p)

            pltpu.sync_copy(ones_v, acc_sh.at[dbuf[p]], add=True)

        return carry

    lax.fori_loop(0, CPT // 2, outer, 0)
    plsc.subcore_barrier()

    pltpu.sync_copy(acc_sh.at[pl.ds(r0, RPT)],
                    out_hbm.at[c, pl.ds(r0, RPT)])


def _make_sc_cnt():
    return functools.partial(
        pl.kernel,
        mesh=_sc_mesh(),
        out_type=jax.ShapeDtypeStruct((NC, NP, H), _f32),
        scratch_types=[
            pltpu.VMEM((CH,), jnp.int32),
            pltpu.VMEM((CH,), jnp.int32),
            pltpu.VMEM((CH, H), _f32),
            pltpu.VMEM_SHARED((NP, H), _f32),
            pltpu.SemaphoreType.DMA,
            pltpu.SemaphoreType.DMA,
        ],
    )(_sc_cnt_body)


def _recip(c):
    r = 1.0 / c
    r = r * (2.0 - c * r)
    r = r * (2.0 - c * r)
    return r


def _layer_body(p_ref, cnt_ref, xp_ref, wl_ref, bl_ref, wr_ref, h_ref):
    inv = _recip(jnp.maximum(cnt_ref[0] + cnt_ref[1], 1.0))
    mean = (p_ref[0] + p_ref[1]) * inv
    h_ref[...] = jnp.maximum(
        jnp.dot(mean, wl_ref[...], preferred_element_type=_f32) +
        bl_ref[...] +
        jnp.dot(xp_ref[...], wr_ref[...], preferred_element_type=_f32), 0.0)


def _pool_body(h_ref, b_ref, wg_ref, bg_ref, w1_ref, b1_ref, w2_ref,
               b2_ref, o_ref):
    h = h_ref[...]
    gate = jnp.dot(h, wg_ref[...], preferred_element_type=_f32) + bg_ref[...]
    mask = b_ref[...] == lax.broadcasted_iota(jnp.int32, (N, G), 1)
    mg = jnp.max(jnp.where(mask, gate, -3e38), axis=0, keepdims=True)
    ew = jnp.where(mask, jnp.exp(gate - mg), 0.0)
    dn = jnp.sum(ew, axis=0, keepdims=True)
    alpha = ew * _recip(jnp.where(dn > 0.0, dn, 1.0))
    pooled = lax.dot_general(alpha, h, (((0,), (0,)), ((), ())),
                             preferred_element_type=_f32,
                             precision=lax.Precision.HIGHEST)
    t = jnp.maximum(
        jnp.dot(pooled, w1_ref[...], preferred_element_type=_f32) +
        b1_ref[...], 0.0)
    o_ref[...] = jnp.dot(t, w2_ref[...], preferred_element_type=_f32) + b2_ref[...]


def _row_spec(width):
    return pl.BlockSpec((BM, width), lambda i: (i, 0))


def _rep_spec(shape):
    nd = len(shape)
    return pl.BlockSpec(shape, lambda i: (0,) * nd)


def _tc_layer(p, cnt2d, xp, wl, bl, wr):
    return pl.pallas_call(
        _layer_body,
        grid=(NB,),
        in_specs=[
            pl.BlockSpec((NC, BM, H), lambda i: (0, i, 0)),
            pl.BlockSpec((NC, BM, 1), lambda i: (0, i, 0)),
            _row_spec(H), _rep_spec((H, H)), _rep_spec((1, H)),
            _rep_spec((H, H)),
        ],
        out_specs=_row_spec(H),
        out_shape=jax.ShapeDtypeStruct((N, H), _f32),
    )(p, cnt2d, xp, wl, bl, wr)


def _tc_pool(h, batch2d, wg, bg, w1, b1, w2, b2):
    return pl.pallas_call(
        _pool_body,
        out_shape=jax.ShapeDtypeStruct((G, 1), _f32),
    )(h, batch2d, wg, bg, w1, b1, w2, b2)


def kernel(x, edge_index, batch, Wl1, bl1, Wr1, Wl2, bl2, Wr2, Wl3, bl3, Wr3,
           Wg, bg, W1, b1, W2, b2):
    pad = EP - E
    src2d = jnp.concatenate(
        [edge_index[0], jnp.zeros((pad,), jnp.int32)]).reshape(EC, CH)
    dst2d = jnp.concatenate(
        [edge_index[1], jnp.full((pad,), N, jnp.int32)]).reshape(EC, CH)
    batch2d = batch.reshape(N, 1)
    zz = jnp.zeros((NP, H), _f32)

    sc_agg = _make_sc_agg()
    cnt2d = _make_sc_cnt()(dst2d, jnp.ones((CH, H), _f32), zz)[:, :, :1]

    p1 = sc_agg(src2d, dst2d, x, zz)
    h1 = _tc_layer(p1, cnt2d, x, Wl1, bl1.reshape(1, H), Wr1)
    p2 = sc_agg(src2d, dst2d, h1, zz)
    h2 = _tc_layer(p2, cnt2d, h1, Wl2, bl2.reshape(1, H), Wr2)
    p3 = sc_agg(src2d, dst2d, h2, zz)
    h3 = _tc_layer(p3, cnt2d, h2, Wl3, bl3.reshape(1, H), Wr3)
    out = _tc_pool(h3, batch2d, Wg, bg.reshape(1, 1),
                   W1, b1.reshape(1, H), W2, b2.reshape(1, 1))
    return out


# speedup vs baseline: 1.6513x; 1.6377x over previous
import functools

import jax
import jax.numpy as jnp
from jax import lax
from jax.experimental import pallas as pl
from jax.experimental.pallas import tpu as pltpu
from jax.experimental.pallas import tpu_sc as plsc

N = 10000
E = 320000
D = 128
H = 128
G = 64

CH = 128
EC = E // CH
NC = 2
NS = 16
NW = NC * NS
NP = 10240
RPT = NP // NS
EPT = E // NS
BM = 2000
NB = N // BM

_f32 = jnp.float32


def _sc_mesh():
    return plsc.VectorSubcoreMesh(core_axis_name="c", subcore_axis_name="s")


def _sc_agg_body(src_hbm, dst_hbm, z_hbm, zz_hbm, out_hbm,
                 src_v, dst_v, rows_v, acc_sh, sem):
    c = lax.axis_index("c")
    s = lax.axis_index("s")
    w = s * NC + c
    r0 = s * RPT

    pltpu.sync_copy(zz_hbm.at[pl.ds(r0, RPT)], acc_sh.at[pl.ds(r0, RPT)])
    plsc.subcore_barrier()

    steps = (EC + NW - 1) // NW

    def body(j, carry):
        cid = w + j * NW

        @pl.when(cid < EC)
        def _():
            pltpu.sync_copy(src_hbm.at[cid], src_v)
            pltpu.sync_copy(dst_hbm.at[cid], dst_v)
            pltpu.async_copy(z_hbm.at[src_v], rows_v, sem).wait()
            pltpu.sync_copy(rows_v, acc_sh.at[dst_v], add=True)

        return carry

    lax.fori_loop(0, steps, body, 0)
    plsc.subcore_barrier()

    pltpu.sync_copy(acc_sh.at[pl.ds(r0, RPT)],
                    out_hbm.at[c, pl.ds(r0, RPT)])


def _make_sc_agg():
    return functools.partial(
        pl.kernel,
        mesh=_sc_mesh(),
        out_type=jax.ShapeDtypeStruct((NC, NP, H), _f32),
        scratch_types=[
            pltpu.VMEM((CH,), jnp.int32),
            pltpu.VMEM((CH,), jnp.int32),
            pltpu.VMEM((CH, H), _f32),
            pltpu.VMEM_SHARED((NP, H), _f32),
            pltpu.SemaphoreType.DMA,
        ],
    )(_sc_agg_body)


def _sc_cnt_body(dst_hbm, ones_hbm, zz_hbm, out_hbm,
                 dst_v, ones_v, acc_sh, sem):
    c = lax.axis_index("c")
    s = lax.axis_index("s")
    w = s * NC + c
    r0 = s * RPT

    pltpu.sync_copy(zz_hbm.at[pl.ds(r0, RPT)], acc_sh.at[pl.ds(r0, RPT)])
    pltpu.sync_copy(ones_hbm, ones_v)
    plsc.subcore_barrier()

    steps = (EC + NW - 1) // NW

    def body(j, carry):
        cid = w + j * NW

        @pl.when(cid < EC)
        def _():
            pltpu.sync_copy(dst_hbm.at[cid], dst_v)
            pltpu.sync_copy(ones_v, acc_sh.at[dst_v], add=True)

        return carry

    lax.fori_loop(0, steps, body, 0)
    plsc.subcore_barrier()

    pltpu.sync_copy(acc_sh.at[pl.ds(r0, RPT)],
                    out_hbm.at[c, pl.ds(r0, RPT)])


def _make_sc_cnt():
    return functools.partial(
        pl.kernel,
        mesh=_sc_mesh(),
        out_type=jax.ShapeDtypeStruct((NC, NP, H), _f32),
        scratch_types=[
            pltpu.VMEM((CH,), jnp.int32),
            pltpu.VMEM((CH, H), _f32),
            pltpu.VMEM_SHARED((NP, H), _f32),
            pltpu.SemaphoreType.DMA,
        ],
    )(_sc_cnt_body)


def _recip(c):
    r = 1.0 / c
    r = r * (2.0 - c * r)
    r = r * (2.0 - c * r)
    return r


def _layer_body(p_ref, cnt_ref, xp_ref, wl_ref, bl_ref, wr_ref, h_ref):
    inv = _recip(jnp.maximum(cnt_ref[0] + cnt_ref[1], 1.0))
    mean = (p_ref[0] + p_ref[1]) * inv
    h_ref[...] = jnp.maximum(
        jnp.dot(mean, wl_ref[...], preferred_element_type=_f32) +
        bl_ref[...] +
        jnp.dot(xp_ref[...], wr_ref[...], preferred_element_type=_f32), 0.0)


def _pool_body(h_ref, b_ref, wg_ref, bg_ref, w1_ref, b1_ref, w2_ref,
               b2_ref, o_ref):
    h = h_ref[...]
    gate = jnp.dot(h, wg_ref[...], preferred_element_type=_f32) + bg_ref[...]
    mask = b_ref[...] == lax.broadcasted_iota(jnp.int32, (N, G), 1)
    mg = jnp.max(jnp.where(mask, gate, -3e38), axis=0, keepdims=True)
    ew = jnp.where(mask, jnp.exp(gate - mg), 0.0)
    dn = jnp.sum(ew, axis=0, keepdims=True)
    alpha = ew * _recip(jnp.where(dn > 0.0, dn, 1.0))
    pooled = lax.dot_general(alpha, h, (((0,), (0,)), ((), ())),
                             preferred_element_type=_f32,
                             precision=lax.Precision.HIGHEST)
    t = jnp.maximum(
        jnp.dot(pooled, w1_ref[...], preferred_element_type=_f32) +
        b1_ref[...], 0.0)
    o_ref[...] = jnp.dot(t, w2_ref[...], preferred_element_type=_f32) + b2_ref[...]


def _row_spec(width):
    return pl.BlockSpec((BM, width), lambda i: (i, 0))


def _rep_spec(shape):
    nd = len(shape)
    return pl.BlockSpec(shape, lambda i: (0,) * nd)


def _tc_layer(p, cnt2d, xp, wl, bl, wr):
    return pl.pallas_call(
        _layer_body,
        grid=(NB,),
        in_specs=[
            pl.BlockSpec((NC, BM, H), lambda i: (0, i, 0)),
            pl.BlockSpec((NC, BM, 1), lambda i: (0, i, 0)),
            _row_spec(H), _rep_spec((H, H)), _rep_spec((1, H)),
            _rep_spec((H, H)),
        ],
        out_specs=_row_spec(H),
        out_shape=jax.ShapeDtypeStruct((N, H), _f32),
    )(p, cnt2d, xp, wl, bl, wr)


def _tc_pool(h, batch2d, wg, bg, w1, b1, w2, b2):
    return pl.pallas_call(
        _pool_body,
        out_shape=jax.ShapeDtypeStruct((G, 1), _f32),
    )(h, batch2d, wg, bg, w1, b1, w2, b2)


def kernel(x, edge_index, batch, Wl1, bl1, Wr1, Wl2, bl2, Wr2, Wl3, bl3, Wr3,
           Wg, bg, W1, b1, W2, b2):
    src2d = edge_index[0].reshape(EC, CH)
    dst2d = edge_index[1].reshape(EC, CH)
    batch2d = batch.reshape(N, 1)
    zz = jnp.zeros((NP, H), _f32)

    sc_agg = _make_sc_agg()
    cnt2d = _make_sc_cnt()(dst2d, jnp.ones((CH, H), _f32), zz)[:, :, :1]

    p1 = sc_agg(src2d, dst2d, x, zz)
    h1 = _tc_layer(p1, cnt2d, x, Wl1, bl1.reshape(1, H), Wr1)
    p2 = sc_agg(src2d, dst2d, h1, zz)
    h2 = _tc_layer(p2, cnt2d, h1, Wl2, bl2.reshape(1, H), Wr2)
    p3 = sc_agg(src2d, dst2d, h2, zz)
    h3 = _tc_layer(p3, cnt2d, h2, Wl3, bl3.reshape(1, H), Wr3)
    out = _tc_pool(h3, batch2d, Wg, bg.reshape(1, 1),
                   W1, b1.reshape(1, H), W2, b2.reshape(1, 1))
    return out


# speedup vs baseline: 1.8403x; 1.1144x over previous
import functools

import jax
import jax.numpy as jnp
from jax import lax
from jax.experimental import pallas as pl
from jax.experimental.pallas import tpu as pltpu
from jax.experimental.pallas import tpu_sc as plsc

N = 10000
E = 320000
D = 128
H = 128
G = 64

CH = 128
EC = E // CH
NC = 2
NS = 16
NW = NC * NS
NP = 10240
RPT = NP // NS
EPT = E // NS
BM = 2000
NB = N // BM

_f32 = jnp.float32


def _sc_mesh():
    return plsc.VectorSubcoreMesh(core_axis_name="c", subcore_axis_name="s")


def _sc_agg_body(pk_hbm, z_hbm, zz_hbm, out_hbm,
                 idx_v, rows_v, acc_sh, sem):
    c = lax.axis_index("c")
    s = lax.axis_index("s")
    w = s * NC + c
    r0 = s * RPT

    pltpu.sync_copy(zz_hbm.at[pl.ds(r0, RPT)], acc_sh.at[pl.ds(r0, RPT)])
    plsc.subcore_barrier()

    steps = (EC + NW - 1) // NW

    def body(j, carry):
        cid = w + j * NW

        @pl.when(cid < EC)
        def _():
            pltpu.sync_copy(pk_hbm.at[cid], idx_v)
            pltpu.async_copy(z_hbm.at[idx_v.at[0]], rows_v, sem).wait()
            pltpu.sync_copy(rows_v, acc_sh.at[idx_v.at[1]], add=True)

        return carry

    lax.fori_loop(0, steps, body, 0)
    plsc.subcore_barrier()

    pltpu.sync_copy(acc_sh.at[pl.ds(r0, RPT)],
                    out_hbm.at[c, pl.ds(r0, RPT)])


def _make_sc_agg():
    return functools.partial(
        pl.kernel,
        mesh=_sc_mesh(),
        out_type=jax.ShapeDtypeStruct((NC, NP, H), _f32),
        scratch_types=[
            pltpu.VMEM((2, CH), jnp.int32),
            pltpu.VMEM((CH, H), _f32),
            pltpu.VMEM_SHARED((NP, H), _f32),
            pltpu.SemaphoreType.DMA,
        ],
    )(_sc_agg_body)


def _sc_cnt_body(dst_hbm, ones_hbm, zz_hbm, out_hbm,
                 dst_v, ones_v, acc_sh, sem):
    c = lax.axis_index("c")
    s = lax.axis_index("s")
    w = s * NC + c
    r0 = s * RPT

    pltpu.sync_copy(zz_hbm.at[pl.ds(r0, RPT)], acc_sh.at[pl.ds(r0, RPT)])
    pltpu.sync_copy(ones_hbm, ones_v)
    plsc.subcore_barrier()

    steps = (EC + NW - 1) // NW

    def body(j, carry):
        cid = w + j * NW

        @pl.when(cid < EC)
        def _():
            pltpu.sync_copy(dst_hbm.at[cid], dst_v)
            pltpu.sync_copy(ones_v, acc_sh.at[dst_v], add=True)

        return carry

    lax.fori_loop(0, steps, body, 0)
    plsc.subcore_barrier()

    pltpu.sync_copy(acc_sh.at[pl.ds(r0, RPT)],
                    out_hbm.at[c, pl.ds(r0, RPT)])


def _make_sc_cnt():
    return functools.partial(
        pl.kernel,
        mesh=_sc_mesh(),
        out_type=jax.ShapeDtypeStruct((NC, NP, H), _f32),
        scratch_types=[
            pltpu.VMEM((CH,), jnp.int32),
            pltpu.VMEM((CH, H), _f32),
            pltpu.VMEM_SHARED((NP, H), _f32),
            pltpu.SemaphoreType.DMA,
        ],
    )(_sc_cnt_body)


def _recip(c):
    r = 1.0 / c
    r = r * (2.0 - c * r)
    r = r * (2.0 - c * r)
    return r


def _layer_body(p_ref, cnt_ref, xp_ref, wl_ref, bl_ref, wr_ref, h_ref):
    inv = _recip(jnp.maximum(cnt_ref[0] + cnt_ref[1], 1.0))
    mean = (p_ref[0] + p_ref[1]) * inv
    h_ref[...] = jnp.maximum(
        jnp.dot(mean, wl_ref[...], preferred_element_type=_f32) +
        bl_ref[...] +
        jnp.dot(xp_ref[...], wr_ref[...], preferred_element_type=_f32), 0.0)


def _pool_body(h_ref, b_ref, wg_ref, bg_ref, w1_ref, b1_ref, w2_ref,
               b2_ref, o_ref):
    h = h_ref[...]
    gate = jnp.dot(h, wg_ref[...], preferred_element_type=_f32) + bg_ref[...]
    mask = b_ref[...] == lax.broadcasted_iota(jnp.int32, (N, G), 1)
    mg = jnp.max(jnp.where(mask, gate, -3e38), axis=0, keepdims=True)
    ew = jnp.where(mask, jnp.exp(gate - mg), 0.0)
    dn = jnp.sum(ew, axis=0, keepdims=True)
    alpha = ew * _recip(jnp.where(dn > 0.0, dn, 1.0))
    pooled = lax.dot_general(alpha, h, (((0,), (0,)), ((), ())),
                             preferred_element_type=_f32,
                             precision=lax.Precision.HIGHEST)
    t = jnp.maximum(
        jnp.dot(pooled, w1_ref[...], preferred_element_type=_f32) +
        b1_ref[...], 0.0)
    o_ref[...] = jnp.dot(t, w2_ref[...], preferred_element_type=_f32) + b2_ref[...]


def _row_spec(width):
    return pl.BlockSpec((BM, width), lambda i: (i, 0))


def _rep_spec(shape):
    nd = len(shape)
    return pl.BlockSpec(shape, lambda i: (0,) * nd)


def _tc_layer(p, cnt2d, xp, wl, bl, wr):
    return pl.pallas_call(
        _layer_body,
        grid=(NB,),
        in_specs=[
            pl.BlockSpec((NC, BM, H), lambda i: (0, i, 0)),
            pl.BlockSpec((NC, BM, 1), lambda i: (0, i, 0)),
            _row_spec(H), _rep_spec((H, H)), _rep_spec((1, H)),
            _rep_spec((H, H)),
        ],
        out_specs=_row_spec(H),
        out_shape=jax.ShapeDtypeStruct((N, H), _f32),
    )(p, cnt2d, xp, wl, bl, wr)


def _tc_pool(h, batch2d, wg, bg, w1, b1, w2, b2):
    return pl.pallas_call(
        _pool_body,
        out_shape=jax.ShapeDtypeStruct((G, 1), _f32),
    )(h, batch2d, wg, bg, w1, b1, w2, b2)


def kernel(x, edge_index, batch, Wl1, bl1, Wr1, Wl2, bl2, Wr2, Wl3, bl3, Wr3,
           Wg, bg, W1, b1, W2, b2):
    src2d = edge_index[0].reshape(EC, CH)
    dst2d = edge_index[1].reshape(EC, CH)
    batch2d = batch.reshape(N, 1)
    zz = jnp.zeros((NP, H), _f32)

    sc_agg = _make_sc_agg()
    cnt2d = _make_sc_cnt()(dst2d, jnp.ones((CH, H), _f32), zz)[:, :, :1]

    pk = jnp.stack([src2d, dst2d], axis=1)
    p1 = sc_agg(pk, x, zz)
    h1 = _tc_layer(p1, cnt2d, x, Wl1, bl1.reshape(1, H), Wr1)
    p2 = sc_agg(pk, h1, zz)
    h2 = _tc_layer(p2, cnt2d, h1, Wl2, bl2.reshape(1, H), Wr2)
    p3 = sc_agg(pk, h2, zz)
    h3 = _tc_layer(p3, cnt2d, h2, Wl3, bl3.reshape(1, H), Wr3)
    out = _tc_pool(h3, batch2d, Wg, bg.reshape(1, 1),
                   W1, b1.reshape(1, H), W2, b2.reshape(1, 1))
    return out
